# baseline (speedup 1.0000x reference)
import functools

import jax
import jax.numpy as jnp
from jax import lax
from jax.experimental import pallas as pl
from jax.experimental.pallas import tpu as pltpu
from jax.experimental.pallas import tpu_sc as plsc

TM = 256
_NC = 2
_NS = 16
_NW = _NC * _NS


def _sc_mesh():
    return plsc.VectorSubcoreMesh(core_axis_name="c", subcore_axis_name="s")


def _sc_gather(x, idx):
    n, d = x.shape
    b = idx.shape[0]
    per_w = b // _NW
    chunk = 64
    n_chunks = per_w // chunk

    @functools.partial(
        pl.kernel,
        out_type=jax.ShapeDtypeStruct((b, d), x.dtype),
        mesh=_sc_mesh(),
        scratch_types=[
            pltpu.VMEM((chunk,), jnp.int32),
            pltpu.VMEM((chunk, d), x.dtype),
            pltpu.SemaphoreType.DMA,
        ],
    )
    def k(x_hbm, idx_hbm, out_hbm, idx_v, rows_v, sem):
        wid = lax.axis_index("s") * _NC + lax.axis_index("c")
        base = wid * per_w
        for c in range(n_chunks):
            off = base + c * chunk
            pltpu.sync_copy(idx_hbm.at[pl.ds(off, chunk)], idx_v)
            pltpu.async_copy(x_hbm.at[idx_v], rows_v, sem).wait()
            pltpu.sync_copy(rows_v, out_hbm.at[pl.ds(off, chunk)])

    return k(x, idx)


def _sc_combine(s, p0, p1):
    _, d = s.shape
    n = p0.shape[0]
    per_w = n // _NW
    chunk = 32
    n_chunks = per_w // chunk
    nvec = d // 16

    @functools.partial(
        pl.kernel,
        out_type=jax.ShapeDtypeStruct((n, d), s.dtype),
        mesh=_sc_mesh(),
        scratch_types=[
            pltpu.VMEM((chunk,), jnp.int32),
            pltpu.VMEM((chunk,), jnp.int32),
            pltpu.VMEM((chunk, d), s.dtype),
            pltpu.VMEM((chunk, d), s.dtype),
            pltpu.SemaphoreType.DMA,
        ],
    )
    def k(s_hbm, p0_hbm, p1_hbm, out_hbm, p0_v, p1_v, buf0, buf1, sem):
        wid = lax.axis_index("s") * _NC + lax.axis_index("c")
        base = wid * per_w
        for c in range(n_chunks):
            off = base + c * chunk
            pltpu.sync_copy(p0_hbm.at[pl.ds(off, chunk)], p0_v)
            pltpu.sync_copy(p1_hbm.at[pl.ds(off, chunk)], p1_v)
            pltpu.async_copy(s_hbm.at[p0_v], buf0, sem).wait()
            pltpu.async_copy(s_hbm.at[p1_v], buf1, sem).wait()

            def row_body(r, _):
                for cc in range(nvec):
                    sl = pl.ds(cc * 16, 16)
                    buf0[r, sl] = buf0[r, sl] + buf1[r, sl]
                return 0

            lax.fori_loop(0, chunk, row_body, 0)
            pltpu.sync_copy(buf0, out_hbm.at[pl.ds(off, chunk)])

    return k(s, p0, p1)


def _cdiv(a, b):
    return (a + b - 1) // b


def _ffn_kernel(gid_ref, mid_ref, s_ref, e_ref, first_ref,
                x_ref, w1_ref, w2_ref, w3_ref, sw_ref, out_ref, *, nj):
    u = pl.program_id(0)
    j = pl.program_id(1) if nj > 1 else 0
    s = s_ref[u]
    e = e_ref[u]
    m = mid_ref[u]
    rows = m * TM + lax.broadcasted_iota(jnp.int32, (TM, 1), 0)
    msk = (rows >= s) & (rows < e)
    xb = x_ref[...].astype(jnp.bfloat16)
    g = jnp.dot(xb, w1_ref[0], preferred_element_type=jnp.float32)
    v = jnp.dot(xb, w2_ref[0], preferred_element_type=jnp.float32)
    h = jax.nn.silu(g) * v
    h = h * sw_ref[...]
    h = jnp.where(msk, h, 0.0)
    contrib = jnp.dot(h.astype(jnp.bfloat16), w3_ref[0],
                      preferred_element_type=jnp.float32)

    is_first = (first_ref[u] != 0) & (j == 0)

    @pl.when(is_first)
    def _():
        out_ref[...] = contrib

    @pl.when(jnp.logical_not(is_first))
    def _():
        out_ref[...] += contrib


def _grouped_ffn(x_sorted, sorted_weights, w1, w2, w3, offsets, *, nj=1,
                 interpret=False):
    m, k = x_sorted.shape
    n_exp, _, f = w1.shape
    tiles_m = m // TM
    lw = tiles_m + n_exp - 1
    tf = f // nj

    s_g = offsets[:-1]
    e_g = offsets[1:]
    t0 = s_g // TM
    t1 = lax.div(e_g + (TM - 1), TM) - 1
    n_g = jnp.where(e_g > s_g, t1 - t0 + 1, 0)
    cum = jnp.cumsum(n_g)
    total = cum[-1]
    u = jnp.arange(lw, dtype=jnp.int32)
    g_u = jnp.searchsorted(cum, u, side="right").astype(jnp.int32)
    valid = u < total
    g_u = jnp.where(valid, jnp.minimum(g_u, n_exp - 1), n_exp - 1)
    unit_start = cum[g_u] - n_g[g_u]
    m_u = jnp.where(valid, t0[g_u] + (u - unit_start), tiles_m - 1)
    s_u = jnp.where(valid, s_g[g_u], 0).astype(jnp.int32)
    e_u = jnp.where(valid, e_g[g_u], 0).astype(jnp.int32)
    first_u = jnp.concatenate(
        [jnp.ones((1,), jnp.int32),
         (m_u[1:] != m_u[:-1]).astype(jnp.int32)])
    m_u = m_u.astype(jnp.int32)

    grid = (lw,) if nj == 1 else (lw, nj)
    if nj == 1:
        xmap = lambda u, *refs: (refs[1][u], 0)
        w12map = lambda u, *refs: (refs[0][u], 0, 0)
        w3map = lambda u, *refs: (refs[0][u], 0, 0)
    else:
        xmap = lambda u, j, *refs: (refs[1][u], 0)
        w12map = lambda u, j, *refs: (refs[0][u], 0, j)
        w3map = lambda u, j, *refs: (refs[0][u], j, 0)

    grid_spec = pltpu.PrefetchScalarGridSpec(
        num_scalar_prefetch=5,
        grid=grid,
        in_specs=[
            pl.BlockSpec((TM, k), xmap),
            pl.BlockSpec((1, k, tf), w12map),
            pl.BlockSpec((1, k, tf), w12map),
            pl.BlockSpec((1, tf, k), w3map),
            pl.BlockSpec((TM, 1), xmap),
        ],
        out_specs=pl.BlockSpec((TM, k), xmap),
    )
    return pl.pallas_call(
        functools.partial(_ffn_kernel, nj=nj),
        grid_spec=grid_spec,
        out_shape=jax.ShapeDtypeStruct((m, k), jnp.float32),
        compiler_params=pltpu.CompilerParams(
            dimension_semantics=("arbitrary",) * len(grid)),
        interpret=interpret,
    )(g_u, m_u, s_u, e_u, first_u,
      x_sorted, w1, w2, w3, sorted_weights.reshape(m, 1))


def kernel(x, expert_indices, expert_weights, w1, w2, w3):
    n_tokens, d_model = x.shape
    top_k = expert_indices.shape[1]
    n_experts = w1.shape[0]
    m = n_tokens * top_k

    flat_experts = expert_indices.reshape(-1).astype(jnp.int32)
    flat_weights = expert_weights.reshape(-1)
    sorted_order = jnp.argsort(flat_experts, stable=True).astype(jnp.int32)
    token_indices = (sorted_order // top_k).astype(jnp.int32)
    sorted_weights = flat_weights[sorted_order]
    counts = jnp.zeros((n_experts,), jnp.int32).at[flat_experts].add(1)
    offsets = jnp.concatenate(
        [jnp.zeros((1,), jnp.int32), jnp.cumsum(counts).astype(jnp.int32)])
    inv = jnp.zeros((m,), jnp.int32).at[sorted_order].set(
        jnp.arange(m, dtype=jnp.int32))

    x_sorted = _sc_gather(x, token_indices)
    s = _grouped_ffn(x_sorted, sorted_weights,
                     w1.astype(jnp.bfloat16), w2.astype(jnp.bfloat16),
                     w3.astype(jnp.bfloat16), offsets, nj=1)
    inv2 = inv.reshape(n_tokens, top_k)
    out = _sc_combine(s, inv2[:, 0], inv2[:, 1])
    return out


# speedup vs baseline: 1.2655x; 1.2655x over previous
import functools

import jax
import jax.numpy as jnp
from jax import lax
from jax.experimental import pallas as pl
from jax.experimental.pallas import tpu as pltpu
from jax.experimental.pallas import tpu_sc as plsc

TM = 128
_NC = 2
_NS = 16
_NW = _NC * _NS


def _sc_mesh():
    return plsc.VectorSubcoreMesh(core_axis_name="c", subcore_axis_name="s")


def _sc_gather(x, idx):
    n, d = x.shape
    b = idx.shape[0]
    per_w = b // _NW
    chunk = 64
    n_chunks = per_w // chunk

    @functools.partial(
        pl.kernel,
        out_type=jax.ShapeDtypeStruct((b, d), x.dtype),
        mesh=_sc_mesh(),
        scratch_types=[
            pltpu.VMEM((chunk,), jnp.int32),
            pltpu.VMEM((chunk, d), x.dtype),
            pltpu.SemaphoreType.DMA,
        ],
    )
    def k(x_hbm, idx_hbm, out_hbm, idx_v, rows_v, sem):
        wid = lax.axis_index("s") * _NC + lax.axis_index("c")
        base = wid * per_w
        for c in range(n_chunks):
            off = base + c * chunk
            pltpu.sync_copy(idx_hbm.at[pl.ds(off, chunk)], idx_v)
            pltpu.async_copy(x_hbm.at[idx_v], rows_v, sem).wait()
            pltpu.sync_copy(rows_v, out_hbm.at[pl.ds(off, chunk)])

    return k(x, idx)


def _sc_combine(s, p0, p1):
    _, d = s.shape
    n = p0.shape[0]
    per_w = n // _NW
    chunk = 32
    n_chunks = per_w // chunk
    nvec = d // 16

    @functools.partial(
        pl.kernel,
        out_type=jax.ShapeDtypeStruct((n, d), s.dtype),
        mesh=_sc_mesh(),
        scratch_types=[
            pltpu.VMEM((chunk,), jnp.int32),
            pltpu.VMEM((chunk,), jnp.int32),
            pltpu.VMEM((chunk, d), s.dtype),
            pltpu.VMEM((chunk, d), s.dtype),
            pltpu.SemaphoreType.DMA,
        ],
    )
    def k(s_hbm, p0_hbm, p1_hbm, out_hbm, p0_v, p1_v, buf0, buf1, sem):
        wid = lax.axis_index("s") * _NC + lax.axis_index("c")
        base = wid * per_w
        for c in range(n_chunks):
            off = base + c * chunk
            pltpu.sync_copy(p0_hbm.at[pl.ds(off, chunk)], p0_v)
            pltpu.sync_copy(p1_hbm.at[pl.ds(off, chunk)], p1_v)
            pltpu.async_copy(s_hbm.at[p0_v], buf0, sem).wait()
            pltpu.async_copy(s_hbm.at[p1_v], buf1, sem).wait()

            def row_body(r, _):
                for cc in range(nvec):
                    sl = pl.ds(cc * 16, 16)
                    buf0[r, sl] = buf0[r, sl] + buf1[r, sl]
                return 0

            lax.fori_loop(0, chunk, row_body, 0)
            pltpu.sync_copy(buf0, out_hbm.at[pl.ds(off, chunk)])

    return k(s, p0, p1)


def _cdiv(a, b):
    return (a + b - 1) // b


def _ffn_kernel(gid_ref, mid_ref, s_ref, e_ref, first_ref,
                x_ref, w1_ref, w2_ref, w3_ref, sw_ref, out_ref, *, nj):
    u = pl.program_id(0)
    j = pl.program_id(1) if nj > 1 else 0
    s = s_ref[u]
    e = e_ref[u]
    m = mid_ref[u]
    rows = m * TM + lax.broadcasted_iota(jnp.int32, (TM, 1), 0)
    msk = (rows >= s) & (rows < e)
    xb = x_ref[...]
    g = jnp.dot(xb, w1_ref[0], preferred_element_type=jnp.float32)
    v = jnp.dot(xb, w2_ref[0], preferred_element_type=jnp.float32)
    h = jax.nn.silu(g) * v
    h = h * sw_ref[...]
    h = jnp.where(msk, h, 0.0)
    contrib = jnp.dot(h, w3_ref[0], preferred_element_type=jnp.float32)

    is_first = (first_ref[u] != 0) & (j == 0)

    @pl.when(is_first)
    def _():
        out_ref[...] = contrib

    @pl.when(jnp.logical_not(is_first))
    def _():
        out_ref[...] += contrib


def _grouped_ffn(x_sorted, sorted_weights, w1, w2, w3, offsets, *, nj=1,
                 interpret=False):
    m, k = x_sorted.shape
    n_exp, _, f = w1.shape
    tiles_m = m // TM
    lw = tiles_m + n_exp - 1
    tf = f // nj

    s_g = offsets[:-1]
    e_g = offsets[1:]
    t0 = s_g // TM
    t1 = lax.div(e_g + (TM - 1), TM) - 1
    n_g = jnp.where(e_g > s_g, t1 - t0 + 1, 0)
    cum = jnp.cumsum(n_g)
    total = cum[-1]
    u = jnp.arange(lw, dtype=jnp.int32)
    g_u = jnp.searchsorted(cum, u, side="right").astype(jnp.int32)
    valid = u < total
    g_u = jnp.where(valid, jnp.minimum(g_u, n_exp - 1), n_exp - 1)
    unit_start = cum[g_u] - n_g[g_u]
    m_u = jnp.where(valid, t0[g_u] + (u - unit_start), tiles_m - 1)
    s_u = jnp.where(valid, s_g[g_u], 0).astype(jnp.int32)
    e_u = jnp.where(valid, e_g[g_u], 0).astype(jnp.int32)
    first_u = jnp.concatenate(
        [jnp.ones((1,), jnp.int32),
         (m_u[1:] != m_u[:-1]).astype(jnp.int32)])
    m_u = m_u.astype(jnp.int32)

    grid = (lw,) if nj == 1 else (lw, nj)
    if nj == 1:
        xmap = lambda u, *refs: (refs[1][u], 0)
        w12map = lambda u, *refs: (refs[0][u], 0, 0)
        w3map = lambda u, *refs: (refs[0][u], 0, 0)
    else:
        xmap = lambda u, j, *refs: (refs[1][u], 0)
        w12map = lambda u, j, *refs: (refs[0][u], 0, j)
        w3map = lambda u, j, *refs: (refs[0][u], j, 0)

    grid_spec = pltpu.PrefetchScalarGridSpec(
        num_scalar_prefetch=5,
        grid=grid,
        in_specs=[
            pl.BlockSpec((TM, k), xmap),
            pl.BlockSpec((1, k, tf), w12map),
            pl.BlockSpec((1, k, tf), w12map),
            pl.BlockSpec((1, tf, k), w3map),
            pl.BlockSpec((TM, 1), xmap),
        ],
        out_specs=pl.BlockSpec((TM, k), xmap),
    )
    return pl.pallas_call(
        functools.partial(_ffn_kernel, nj=nj),
        grid_spec=grid_spec,
        out_shape=jax.ShapeDtypeStruct((m, k), jnp.float32),
        compiler_params=pltpu.CompilerParams(
            dimension_semantics=("arbitrary",) * len(grid)),
        interpret=interpret,
    )(g_u, m_u, s_u, e_u, first_u,
      x_sorted, w1, w2, w3, sorted_weights.reshape(m, 1))


def kernel(x, expert_indices, expert_weights, w1, w2, w3):
    n_tokens, d_model = x.shape
    top_k = expert_indices.shape[1]
    n_experts = w1.shape[0]
    m = n_tokens * top_k

    flat_experts = expert_indices.reshape(-1).astype(jnp.int32)
    flat_weights = expert_weights.reshape(-1)
    sorted_order = jnp.argsort(flat_experts, stable=True).astype(jnp.int32)
    token_indices = (sorted_order // top_k).astype(jnp.int32)
    sorted_weights = flat_weights[sorted_order]
    counts = jnp.zeros((n_experts,), jnp.int32).at[flat_experts].add(1)
    offsets = jnp.concatenate(
        [jnp.zeros((1,), jnp.int32), jnp.cumsum(counts).astype(jnp.int32)])
    inv = jnp.zeros((m,), jnp.int32).at[sorted_order].set(
        jnp.arange(m, dtype=jnp.int32))

    x_sorted = _sc_gather(x, token_indices)
    s = _grouped_ffn(x_sorted, sorted_weights, w1, w2, w3, offsets, nj=1)
    inv2 = inv.reshape(n_tokens, top_k)
    out = _sc_combine(s, inv2[:, 0], inv2[:, 1])
    return out


# speedup vs baseline: 1.3505x; 1.0672x over previous
import functools

import jax
import jax.numpy as jnp
from jax import lax
from jax.experimental import pallas as pl
from jax.experimental.pallas import tpu as pltpu
from jax.experimental.pallas import tpu_sc as plsc

TM = 256
_NC = 2
_NS = 16
_NW = _NC * _NS


def _sc_mesh():
    return plsc.VectorSubcoreMesh(core_axis_name="c", subcore_axis_name="s")


def _sc_gather(x, idx):
    n, d = x.shape
    b = idx.shape[0]
    per_w = b // _NW
    chunk = 32
    n_chunks = per_w // chunk

    @functools.partial(
        pl.kernel,
        out_type=jax.ShapeDtypeStruct((b, d), x.dtype),
        mesh=_sc_mesh(),
        scratch_types=[
            pltpu.VMEM((per_w,), jnp.int32),
            pltpu.VMEM((2, chunk, d), x.dtype),
            pltpu.SemaphoreType.DMA,
            pltpu.SemaphoreType.DMA,
            pltpu.SemaphoreType.DMA,
            pltpu.SemaphoreType.DMA,
        ],
    )
    def k(x_hbm, idx_hbm, out_hbm, idx_v, rows_v, g0, g1, s0, s1):
        wid = lax.axis_index("s") * _NC + lax.axis_index("c")
        base = wid * per_w
        gsem = (g0, g1)
        ssem = (s0, s1)
        pltpu.sync_copy(idx_hbm.at[pl.ds(base, per_w)], idx_v)
        gathers = [None] * n_chunks
        stores = [None] * n_chunks
        gathers[0] = pltpu.async_copy(
            x_hbm.at[idx_v.at[pl.ds(0, chunk)]], rows_v.at[0], gsem[0])
        for c in range(n_chunks):
            sl = c & 1
            if c + 1 < n_chunks:
                if c - 1 >= 0:
                    stores[c - 1].wait()
                gathers[c + 1] = pltpu.async_copy(
                    x_hbm.at[idx_v.at[pl.ds((c + 1) * chunk, chunk)]],
                    rows_v.at[1 - sl], gsem[1 - sl])
            gathers[c].wait()
            stores[c] = pltpu.async_copy(
                rows_v.at[sl], out_hbm.at[pl.ds(base + c * chunk, chunk)],
                ssem[sl])
        stores[n_chunks - 2].wait()
        stores[n_chunks - 1].wait()

    return k(x, idx)


def _sc_combine(s, p0, p1):
    _, d = s.shape
    n = p0.shape[0]
    per_w = n // _NW
    chunk = 16
    n_chunks = per_w // chunk
    nvec = d // 16

    @functools.partial(
        pl.kernel,
        out_type=jax.ShapeDtypeStruct((n, d), s.dtype),
        mesh=_sc_mesh(),
        scratch_types=[
            pltpu.VMEM((per_w,), jnp.int32),
            pltpu.VMEM((per_w,), jnp.int32),
            pltpu.VMEM((2, chunk, d), s.dtype),
            pltpu.VMEM((2, chunk, d), s.dtype),
            pltpu.SemaphoreType.DMA,
            pltpu.SemaphoreType.DMA,
            pltpu.SemaphoreType.DMA,
            pltpu.SemaphoreType.DMA,
        ],
    )
    def k(s_hbm, p0_hbm, p1_hbm, out_hbm, p0_v, p1_v, buf0, buf1,
          g0, g1, s0, s1):
        wid = lax.axis_index("s") * _NC + lax.axis_index("c")
        base = wid * per_w
        gsem = (g0, g1)
        ssem = (s0, s1)
        pltpu.sync_copy(p0_hbm.at[pl.ds(base, per_w)], p0_v)
        pltpu.sync_copy(p1_hbm.at[pl.ds(base, per_w)], p1_v)

        def start(c, sl):
            ds = pl.ds(c * chunk, chunk)
            a = pltpu.async_copy(s_hbm.at[p0_v.at[ds]], buf0.at[sl], gsem[sl])
            b = pltpu.async_copy(s_hbm.at[p1_v.at[ds]], buf1.at[sl], gsem[sl])
            return (a, b)

        gathers = [None] * n_chunks
        stores = [None] * n_chunks
        gathers[0] = start(0, 0)

        for c in range(n_chunks):
            sl = c & 1
            if c + 1 < n_chunks:
                if c - 1 >= 0:
                    stores[c - 1].wait()
                gathers[c + 1] = start(c + 1, 1 - sl)
            gathers[c][0].wait()
            gathers[c][1].wait()

            def row_body(r, _):
                for cc in range(nvec):
                    csl = pl.ds(cc * 16, 16)
                    buf0[sl, r, csl] = buf0[sl, r, csl] + buf1[sl, r, csl]
                return 0

            lax.fori_loop(0, chunk, row_body, 0)
            stores[c] = pltpu.async_copy(
                buf0.at[sl], out_hbm.at[pl.ds(base + c * chunk, chunk)],
                ssem[sl])
        stores[n_chunks - 2].wait()
        stores[n_chunks - 1].wait()

    return k(s, p0, p1)


def _cdiv(a, b):
    return (a + b - 1) // b


def _ffn_kernel(gid_ref, mid_ref, s_ref, e_ref, first_ref,
                x_ref, w1_ref, w2_ref, w3_ref, sw_ref, out_ref, *, nj):
    u = pl.program_id(0)
    j = pl.program_id(1) if nj > 1 else 0
    s = s_ref[u]
    e = e_ref[u]
    m = mid_ref[u]
    rows = m * TM + lax.broadcasted_iota(jnp.int32, (TM, 1), 0)
    msk = (rows >= s) & (rows < e)
    xb = x_ref[...]
    g = jnp.dot(xb, w1_ref[0], preferred_element_type=jnp.float32)
    v = jnp.dot(xb, w2_ref[0], preferred_element_type=jnp.float32)
    h = jax.nn.silu(g) * v
    h = h * sw_ref[...]
    h = jnp.where(msk, h, 0.0)
    contrib = jnp.dot(h, w3_ref[0], preferred_element_type=jnp.float32)

    is_first = (first_ref[u] != 0) & (j == 0)

    @pl.when(is_first)
    def _():
        out_ref[...] = contrib

    @pl.when(jnp.logical_not(is_first))
    def _():
        out_ref[...] += contrib


def _grouped_ffn(x_sorted, sorted_weights, w1, w2, w3, offsets, *, nj=1,
                 interpret=False):
    m, k = x_sorted.shape
    n_exp, _, f = w1.shape
    tiles_m = m // TM
    lw = tiles_m + n_exp - 1
    tf = f // nj

    s_g = offsets[:-1]
    e_g = offsets[1:]
    t0 = s_g // TM
    t1 = lax.div(e_g + (TM - 1), TM) - 1
    n_g = jnp.where(e_g > s_g, t1 - t0 + 1, 0)
    cum = jnp.cumsum(n_g)
    total = cum[-1]
    u = jnp.arange(lw, dtype=jnp.int32)
    g_u = jnp.searchsorted(cum, u, side="right").astype(jnp.int32)
    valid = u < total
    g_u = jnp.where(valid, jnp.minimum(g_u, n_exp - 1), n_exp - 1)
    unit_start = cum[g_u] - n_g[g_u]
    m_u = jnp.where(valid, t0[g_u] + (u - unit_start), tiles_m - 1)
    s_u = jnp.where(valid, s_g[g_u], 0).astype(jnp.int32)
    e_u = jnp.where(valid, e_g[g_u], 0).astype(jnp.int32)
    first_u = jnp.concatenate(
        [jnp.ones((1,), jnp.int32),
         (m_u[1:] != m_u[:-1]).astype(jnp.int32)])
    m_u = m_u.astype(jnp.int32)

    grid = (lw,) if nj == 1 else (lw, nj)
    if nj == 1:
        xmap = lambda u, *refs: (refs[1][u], 0)
        w12map = lambda u, *refs: (refs[0][u], 0, 0)
        w3map = lambda u, *refs: (refs[0][u], 0, 0)
    else:
        xmap = lambda u, j, *refs: (refs[1][u], 0)
        w12map = lambda u, j, *refs: (refs[0][u], 0, j)
        w3map = lambda u, j, *refs: (refs[0][u], j, 0)

    grid_spec = pltpu.PrefetchScalarGridSpec(
        num_scalar_prefetch=5,
        grid=grid,
        in_specs=[
            pl.BlockSpec((TM, k), xmap),
            pl.BlockSpec((1, k, tf), w12map),
            pl.BlockSpec((1, k, tf), w12map),
            pl.BlockSpec((1, tf, k), w3map),
            pl.BlockSpec((TM, 1), xmap),
        ],
        out_specs=pl.BlockSpec((TM, k), xmap),
    )
    return pl.pallas_call(
        functools.partial(_ffn_kernel, nj=nj),
        grid_spec=grid_spec,
        out_shape=jax.ShapeDtypeStruct((m, k), jnp.float32),
        compiler_params=pltpu.CompilerParams(
            dimension_semantics=("arbitrary",) * len(grid)),
        interpret=interpret,
    )(g_u, m_u, s_u, e_u, first_u,
      x_sorted, w1, w2, w3, sorted_weights.reshape(m, 1))


def kernel(x, expert_indices, expert_weights, w1, w2, w3):
    n_tokens, d_model = x.shape
    top_k = expert_indices.shape[1]
    n_experts = w1.shape[0]
    m = n_tokens * top_k

    flat_experts = expert_indices.reshape(-1).astype(jnp.int32)
    flat_weights = expert_weights.reshape(-1)
    sorted_order = jnp.argsort(flat_experts, stable=True).astype(jnp.int32)
    token_indices = (sorted_order // top_k).astype(jnp.int32)
    sorted_weights = flat_weights[sorted_order]
    counts = jnp.zeros((n_experts,), jnp.int32).at[flat_experts].add(1)
    offsets = jnp.concatenate(
        [jnp.zeros((1,), jnp.int32), jnp.cumsum(counts).astype(jnp.int32)])
    inv = jnp.zeros((m,), jnp.int32).at[sorted_order].set(
        jnp.arange(m, dtype=jnp.int32))

    x_sorted = _sc_gather(x, token_indices)
    s = _grouped_ffn(x_sorted, sorted_weights, w1, w2, w3, offsets, nj=1)
    inv2 = inv.reshape(n_tokens, top_k)
    out = _sc_combine(s, inv2[:, 0], inv2[:, 1])
    return out


# speedup vs baseline: 1.3755x; 1.0185x over previous
import functools

import jax
import jax.numpy as jnp
from jax import lax
from jax.experimental import pallas as pl
from jax.experimental.pallas import tpu as pltpu
from jax.experimental.pallas import tpu_sc as plsc

TM = 512
_NC = 2
_NS = 16
_NW = _NC * _NS


def _sc_mesh():
    return plsc.VectorSubcoreMesh(core_axis_name="c", subcore_axis_name="s")


def _sc_gather(x, idx):
    n, d = x.shape
    b = idx.shape[0]
    per_w = b // _NW
    chunk = 32
    n_chunks = per_w // chunk

    @functools.partial(
        pl.kernel,
        out_type=jax.ShapeDtypeStruct((b, d), x.dtype),
        mesh=_sc_mesh(),
        scratch_types=[
            pltpu.VMEM((per_w,), jnp.int32),
            pltpu.VMEM((2, chunk, d), x.dtype),
            pltpu.SemaphoreType.DMA,
            pltpu.SemaphoreType.DMA,
            pltpu.SemaphoreType.DMA,
            pltpu.SemaphoreType.DMA,
        ],
    )
    def k(x_hbm, idx_hbm, out_hbm, idx_v, rows_v, g0, g1, s0, s1):
        wid = lax.axis_index("s") * _NC + lax.axis_index("c")
        base = wid * per_w
        gsem = (g0, g1)
        ssem = (s0, s1)
        pltpu.sync_copy(idx_hbm.at[pl.ds(base, per_w)], idx_v)
        gathers = [None] * n_chunks
        stores = [None] * n_chunks
        gathers[0] = pltpu.async_copy(
            x_hbm.at[idx_v.at[pl.ds(0, chunk)]], rows_v.at[0], gsem[0])
        for c in range(n_chunks):
            sl = c & 1
            if c + 1 < n_chunks:
                if c - 1 >= 0:
                    stores[c - 1].wait()
                gathers[c + 1] = pltpu.async_copy(
                    x_hbm.at[idx_v.at[pl.ds((c + 1) * chunk, chunk)]],
                    rows_v.at[1 - sl], gsem[1 - sl])
            gathers[c].wait()
            stores[c] = pltpu.async_copy(
                rows_v.at[sl], out_hbm.at[pl.ds(base + c * chunk, chunk)],
                ssem[sl])
        stores[n_chunks - 2].wait()
        stores[n_chunks - 1].wait()

    return k(x, idx)


def _sc_combine(s, p0, p1):
    _, d = s.shape
    n = p0.shape[0]
    per_w = n // _NW
    chunk = 16
    n_chunks = per_w // chunk
    nvec = d // 16

    @functools.partial(
        pl.kernel,
        out_type=jax.ShapeDtypeStruct((n, d), s.dtype),
        mesh=_sc_mesh(),
        scratch_types=[
            pltpu.VMEM((per_w,), jnp.int32),
            pltpu.VMEM((per_w,), jnp.int32),
            pltpu.VMEM((2, chunk, d), s.dtype),
            pltpu.VMEM((2, chunk, d), s.dtype),
            pltpu.SemaphoreType.DMA,
            pltpu.SemaphoreType.DMA,
            pltpu.SemaphoreType.DMA,
            pltpu.SemaphoreType.DMA,
        ],
    )
    def k(s_hbm, p0_hbm, p1_hbm, out_hbm, p0_v, p1_v, buf0, buf1,
          g0, g1, s0, s1):
        wid = lax.axis_index("s") * _NC + lax.axis_index("c")
        base = wid * per_w
        gsem = (g0, g1)
        ssem = (s0, s1)
        pltpu.sync_copy(p0_hbm.at[pl.ds(base, per_w)], p0_v)
        pltpu.sync_copy(p1_hbm.at[pl.ds(base, per_w)], p1_v)

        def start(c, sl):
            ds = pl.ds(c * chunk, chunk)
            a = pltpu.async_copy(s_hbm.at[p0_v.at[ds]], buf0.at[sl], gsem[sl])
            b = pltpu.async_copy(s_hbm.at[p1_v.at[ds]], buf1.at[sl], gsem[sl])
            return (a, b)

        gathers = [None] * n_chunks
        stores = [None] * n_chunks
        gathers[0] = start(0, 0)

        for c in range(n_chunks):
            sl = c & 1
            if c + 1 < n_chunks:
                if c - 1 >= 0:
                    stores[c - 1].wait()
                gathers[c + 1] = start(c + 1, 1 - sl)
            gathers[c][0].wait()
            gathers[c][1].wait()

            def row_body(r, _):
                for cc in range(nvec):
                    csl = pl.ds(cc * 16, 16)
                    buf0[sl, r, csl] = buf0[sl, r, csl] + buf1[sl, r, csl]
                return 0

            lax.fori_loop(0, chunk, row_body, 0)
            stores[c] = pltpu.async_copy(
                buf0.at[sl], out_hbm.at[pl.ds(base + c * chunk, chunk)],
                ssem[sl])
        stores[n_chunks - 2].wait()
        stores[n_chunks - 1].wait()

    return k(s, p0, p1)


def _cdiv(a, b):
    return (a + b - 1) // b


def _ffn_kernel(gid_ref, mid_ref, s_ref, e_ref, first_ref,
                x_ref, w1_ref, w2_ref, w3_ref, sw_ref, out_ref, *, nj):
    u = pl.program_id(0)
    j = pl.program_id(1) if nj > 1 else 0
    s = s_ref[u]
    e = e_ref[u]
    m = mid_ref[u]
    rows = m * TM + lax.broadcasted_iota(jnp.int32, (TM, 1), 0)
    msk = (rows >= s) & (rows < e)
    xb = x_ref[...]
    g = jnp.dot(xb, w1_ref[0], preferred_element_type=jnp.float32)
    v = jnp.dot(xb, w2_ref[0], preferred_element_type=jnp.float32)
    h = jax.nn.silu(g) * v
    h = h * sw_ref[...]
    h = jnp.where(msk, h, 0.0)
    contrib = jnp.dot(h, w3_ref[0], preferred_element_type=jnp.float32)

    is_first = (first_ref[u] != 0) & (j == 0)

    @pl.when(is_first)
    def _():
        out_ref[...] = contrib

    @pl.when(jnp.logical_not(is_first))
    def _():
        out_ref[...] += contrib


def _grouped_ffn(x_sorted, sorted_weights, w1, w2, w3, offsets, *, nj=1,
                 interpret=False):
    m, k = x_sorted.shape
    n_exp, _, f = w1.shape
    tiles_m = m // TM
    lw = tiles_m + n_exp - 1
    tf = f // nj

    s_g = offsets[:-1]
    e_g = offsets[1:]
    t0 = s_g // TM
    t1 = lax.div(e_g + (TM - 1), TM) - 1
    n_g = jnp.where(e_g > s_g, t1 - t0 + 1, 0)
    cum = jnp.cumsum(n_g)
    total = cum[-1]
    u = jnp.arange(lw, dtype=jnp.int32)
    g_u = jnp.searchsorted(cum, u, side="right").astype(jnp.int32)
    valid = u < total
    g_u = jnp.where(valid, jnp.minimum(g_u, n_exp - 1), n_exp - 1)
    unit_start = cum[g_u] - n_g[g_u]
    m_u = jnp.where(valid, t0[g_u] + (u - unit_start), tiles_m - 1)
    s_u = jnp.where(valid, s_g[g_u], 0).astype(jnp.int32)
    e_u = jnp.where(valid, e_g[g_u], 0).astype(jnp.int32)
    first_u = jnp.concatenate(
        [jnp.ones((1,), jnp.int32),
         (m_u[1:] != m_u[:-1]).astype(jnp.int32)])
    m_u = m_u.astype(jnp.int32)

    grid = (lw,) if nj == 1 else (lw, nj)
    if nj == 1:
        xmap = lambda u, *refs: (refs[1][u], 0)
        w12map = lambda u, *refs: (refs[0][u], 0, 0)
        w3map = lambda u, *refs: (refs[0][u], 0, 0)
    else:
        xmap = lambda u, j, *refs: (refs[1][u], 0)
        w12map = lambda u, j, *refs: (refs[0][u], 0, j)
        w3map = lambda u, j, *refs: (refs[0][u], j, 0)

    grid_spec = pltpu.PrefetchScalarGridSpec(
        num_scalar_prefetch=5,
        grid=grid,
        in_specs=[
            pl.BlockSpec((TM, k), xmap),
            pl.BlockSpec((1, k, tf), w12map),
            pl.BlockSpec((1, k, tf), w12map),
            pl.BlockSpec((1, tf, k), w3map),
            pl.BlockSpec((TM, 1), xmap),
        ],
        out_specs=pl.BlockSpec((TM, k), xmap),
    )
    return pl.pallas_call(
        functools.partial(_ffn_kernel, nj=nj),
        grid_spec=grid_spec,
        out_shape=jax.ShapeDtypeStruct((m, k), jnp.float32),
        compiler_params=pltpu.CompilerParams(
            dimension_semantics=("arbitrary",) * len(grid),
            vmem_limit_bytes=100 * 1024 * 1024),
        interpret=interpret,
    )(g_u, m_u, s_u, e_u, first_u,
      x_sorted, w1, w2, w3, sorted_weights.reshape(m, 1))


def kernel(x, expert_indices, expert_weights, w1, w2, w3):
    n_tokens, d_model = x.shape
    top_k = expert_indices.shape[1]
    n_experts = w1.shape[0]
    m = n_tokens * top_k

    flat_experts = expert_indices.reshape(-1).astype(jnp.int32)
    flat_weights = expert_weights.reshape(-1)
    sorted_order = jnp.argsort(flat_experts, stable=True).astype(jnp.int32)
    token_indices = (sorted_order // top_k).astype(jnp.int32)
    sorted_weights = flat_weights[sorted_order]
    counts = jnp.zeros((n_experts,), jnp.int32).at[flat_experts].add(1)
    offsets = jnp.concatenate(
        [jnp.zeros((1,), jnp.int32), jnp.cumsum(counts).astype(jnp.int32)])
    inv = jnp.zeros((m,), jnp.int32).at[sorted_order].set(
        jnp.arange(m, dtype=jnp.int32))

    x_sorted = _sc_gather(x, token_indices)
    s = _grouped_ffn(x_sorted, sorted_weights, w1, w2, w3, offsets, nj=1)
    inv2 = inv.reshape(n_tokens, top_k)
    out = _sc_combine(s, inv2[:, 0], inv2[:, 1])
    return out
